# baseline (speedup 1.0000x reference)
import functools

import jax
import jax.numpy as jnp
from jax import lax
from jax.experimental import pallas as pl
from jax.experimental.pallas import tpu as pltpu
from jax.experimental.pallas import tpu_sc as plsc

_LAYER = 3
_D = 128
_NC, _NS = 2, 16
_NW = _NC * _NS


def _hswish(x):
    return x * jnp.clip(x + 3.0, 0.0, 6.0) / 6.0


def _mm_body(act, nx, *refs):
    in_refs = refs[:nx]
    w_refs = refs[nx:2 * nx]
    b_ref = refs[2 * nx]
    o_ref = refs[2 * nx + 1]
    acc = b_ref[...].astype(jnp.float32)
    for x_ref, w_ref in zip(in_refs, w_refs):
        acc = acc + jnp.dot(x_ref[...], w_ref[...],
                            preferred_element_type=jnp.float32)
    if act == "hswish":
        acc = _hswish(acc)
    elif act == "sigmoid":
        acc = jax.nn.sigmoid(acc)
    o_ref[...] = acc


def _mm_fused(xs, ws, bias, act, block_rows=2000):
    rows = xs[0].shape[0]
    grid = (rows // block_rows,)
    nx = len(xs)
    in_specs = (
        [pl.BlockSpec((block_rows, x.shape[1]), lambda i: (i, 0)) for x in xs]
        + [pl.BlockSpec(w.shape, lambda i: (0, 0)) for w in ws]
        + [pl.BlockSpec((1, _D), lambda i: (0, 0))]
    )
    return pl.pallas_call(
        functools.partial(_mm_body, act, nx),
        grid=grid,
        in_specs=in_specs,
        out_specs=pl.BlockSpec((block_rows, _D), lambda i: (i, 0)),
        out_shape=jax.ShapeDtypeStruct((rows, _D), jnp.float32),
    )(*xs, *ws, bias.reshape(1, _D))


def _rki_body(pre_ref, mk_ref, w_ref, o_ref):
    mk = mk_ref[...]
    r = jax.nn.sigmoid(pre_ref[...] + jnp.dot(mk, w_ref[...],
                                              preferred_element_type=jnp.float32))
    o_ref[...] = r * mk


def _rki_fused(pre, mk, w, block_rows=2000):
    rows = pre.shape[0]
    return pl.pallas_call(
        _rki_body,
        grid=(rows // block_rows,),
        in_specs=[
            pl.BlockSpec((block_rows, _D), lambda i: (i, 0)),
            pl.BlockSpec((block_rows, _D), lambda i: (i, 0)),
            pl.BlockSpec((_D, _D), lambda i: (0, 0)),
        ],
        out_specs=pl.BlockSpec((block_rows, _D), lambda i: (i, 0)),
        out_shape=jax.ShapeDtypeStruct((rows, _D), jnp.float32),
    )(pre, mk, w)


def _bond_upd_body(pre_z_ref, pre_m_ref, s_ref, r_ref, wz_ref, uw_ref, o_ref):
    s = s_ref[...]
    z = jax.nn.sigmoid(pre_z_ref[...] + jnp.dot(s, wz_ref[...],
                                                preferred_element_type=jnp.float32))
    m = jnp.tanh(pre_m_ref[...] + jnp.dot(r_ref[...], uw_ref[...],
                                          preferred_element_type=jnp.float32))
    o_ref[...] = (1.0 - z) * s + z * m


def _bond_upd(pre_z, pre_m, s, r, wz, uw, block_rows=2000):
    rows = pre_z.shape[0]
    bs = lambda: pl.BlockSpec((block_rows, _D), lambda i: (i, 0))
    return pl.pallas_call(
        _bond_upd_body,
        grid=(rows // block_rows,),
        in_specs=[bs(), bs(), bs(), bs(),
                  pl.BlockSpec((_D, _D), lambda i: (0, 0)),
                  pl.BlockSpec((_D, _D), lambda i: (0, 0))],
        out_specs=bs(),
        out_shape=jax.ShapeDtypeStruct((rows, _D), jnp.float32),
    )(pre_z, pre_m, s, r, wz, uw)


def _node_upd_body(pre_ref, mn_ref, ag_ref, u1_ref, u2_ref, o_ref):
    acc = pre_ref[...]
    acc = acc + jnp.dot(mn_ref[...], u1_ref[...], preferred_element_type=jnp.float32)
    acc = acc + jnp.dot(ag_ref[...], u2_ref[...], preferred_element_type=jnp.float32)
    o_ref[...] = _hswish(acc)


def _node_upd(pre_n, mn, aggr, u1, u2, block_rows=2000):
    rows = pre_n.shape[0]
    bs = lambda: pl.BlockSpec((block_rows, _D), lambda i: (i, 0))
    return pl.pallas_call(
        _node_upd_body,
        grid=(rows // block_rows,),
        in_specs=[bs(), bs(), bs(),
                  pl.BlockSpec((_D, _D), lambda i: (0, 0)),
                  pl.BlockSpec((_D, _D), lambda i: (0, 0))],
        out_specs=bs(),
        out_shape=jax.ShapeDtypeStruct((rows, _D), jnp.float32),
    )(pre_n, mn, aggr, u1, u2)


def _sc_gather(table, idx):
    K = idx.shape[0]
    D = table.shape[1]
    PW = K // _NW
    assert K % _NW == 0 and PW % 8 == 0, (K, PW)
    CH = min(128, PW)
    NFULL = PW // CH
    TAIL = PW - NFULL * CH
    assert TAIL % 8 == 0

    mesh = plsc.VectorSubcoreMesh(core_axis_name="c", subcore_axis_name="s")

    @functools.partial(
        pl.kernel, mesh=mesh,
        out_type=jax.ShapeDtypeStruct((K, D), jnp.float32),
        scratch_types=[
            pltpu.VMEM((PW,), jnp.int32),
            pltpu.VMEM((2, CH, D), jnp.float32),
            pltpu.SemaphoreType.DMA,
            pltpu.SemaphoreType.DMA,
        ],
    )
    def k(table_hbm, idx_hbm, out_hbm, idx_v, rows_v, sem0, sem1):
        wid = lax.axis_index("s") * _NC + lax.axis_index("c")
        base = wid * PW
        pltpu.sync_copy(idx_hbm.at[pl.ds(base, PW)], idx_v)
        sems = (sem0, sem1)

        def fire(c, b):
            pltpu.async_copy(table_hbm.at[idx_v.at[pl.ds(c * CH, CH)]],
                             rows_v.at[b], sems[b])

        def wait_write(c, b):
            pltpu.make_async_copy(
                table_hbm.at[idx_v.at[pl.ds(c * CH, CH)]],
                rows_v.at[b], sems[b]).wait()
            pltpu.sync_copy(rows_v.at[b],
                            out_hbm.at[pl.ds(base + c * CH, CH)])

        fire(0, 0)
        for c in range(1, NFULL):
            fire(c, c & 1)
            wait_write(c - 1, (c - 1) & 1)
        wait_write(NFULL - 1, (NFULL - 1) & 1)
        if TAIL:
            pltpu.async_copy(
                table_hbm.at[idx_v.at[pl.ds(NFULL * CH, TAIL)]],
                rows_v.at[1, pl.ds(0, TAIL)], sem1).wait()
            pltpu.sync_copy(rows_v.at[1, pl.ds(0, TAIL)],
                            out_hbm.at[pl.ds(base + NFULL * CH, TAIL)])

    return k(table, idx)


def _gather_rows(table, idx):
    return _sc_gather(table, idx)


_SEG_R = 8192
_CH = 64


def _sc_segsum_gather(table, pos, seg, num_segments):
    K = seg.shape[0]
    D = table.shape[1]
    assert D == _D
    R = min(_SEG_R, ((num_segments + 4095) // 4096) * 2048)
    NR = (num_segments + R - 1) // R
    S_pad = NR * R
    R16 = R // 16
    NZCH = R16 // 128
    assert R16 % 128 == 0
    PS = K // 16
    NG = PS // 16
    assert K % 256 == 0
    LCAP = PS + 144
    KMAX = (NR + 1) // 2

    zeros_blk = jnp.zeros((128, _D), jnp.float32)
    mesh = plsc.VectorSubcoreMesh(core_axis_name="c", subcore_axis_name="s")

    @functools.partial(
        pl.kernel, mesh=mesh,
        compiler_params=pltpu.CompilerParams(needs_layout_passes=False),
        out_type=jax.ShapeDtypeStruct((S_pad, D), jnp.float32),
        scratch_types=[
            pltpu.VMEM((PS,), jnp.int32),
            pltpu.VMEM((PS,), jnp.int32),
            pltpu.VMEM((LCAP,), jnp.int32),
            pltpu.VMEM((LCAP,), jnp.int32),
            pltpu.VMEM((2, _CH), jnp.int32),
            pltpu.VMEM((2, _CH, D), jnp.float32),
            pltpu.VMEM_SHARED((_SEG_R + 8, _D), jnp.float32),
            pltpu.SemaphoreType.DMA,
            pltpu.SemaphoreType.DMA,
            pltpu.SemaphoreType.DMA,
        ],
    )
    def k(table_hbm, pos_hbm, seg_hbm, zeros_hbm, out_hbm,
          seg_v, pos_v, pos_l, loc_l, loc2d, rows_v, acc,
          sem0, sem1, semz):
        cid = lax.axis_index("c")
        tid = lax.axis_index("s")
        ebase = tid * PS
        pltpu.sync_copy(seg_hbm.at[pl.ds(ebase, PS)], seg_v)
        pltpu.sync_copy(pos_hbm.at[pl.ds(ebase, PS)], pos_v)
        gsem = (sem0, sem1)

        for kk in range(KMAX):
            rid = kk * 2 + cid

            @pl.when(rid < NR)
            def _range():
                lo = rid * R
                zh = [pltpu.async_copy(
                    zeros_hbm, acc.at[pl.ds(tid * R16 + zc * 128, 128)],
                    semz) for zc in range(NZCH)]

                lane = lax.iota(jnp.int32, 16)
                trash = jnp.full((16,), PS + 128, jnp.int32) + lane

                def scan_body(g, cnt_vec):
                    sg = seg_v[pl.ds(g * 16, 16)]
                    m = (sg >= lo) & (sg < lo + R)
                    pref = plsc.cumsum(m.astype(jnp.int32))
                    offs = jnp.where(m, cnt_vec + pref - 1, trash)
                    plsc.store_scatter(pos_l, [offs],
                                       pos_v[pl.ds(g * 16, 16)])
                    plsc.store_scatter(loc_l, [offs], sg - lo)
                    return cnt_vec + plsc.all_reduce_population_count(m)

                cnt_vec = lax.fori_loop(0, NG, scan_body,
                                        jnp.zeros((16,), jnp.int32))
                for g in range(_CH // 16):
                    pad_off = cnt_vec + g * 16 + lane
                    plsc.store_scatter(pos_l, [pad_off],
                                       jnp.zeros((16,), jnp.int32))
                    plsc.store_scatter(loc_l, [pad_off],
                                       jnp.full((16,), R, jnp.int32))
                cnt = jnp.max(cnt_vec, axis=0)
                nch = (cnt + _CH - 1) // _CH
                for h in zh:
                    h.wait()
                plsc.subcore_barrier()

                def fire_g(j, b):
                    pltpu.async_copy(
                        table_hbm.at[pos_l.at[pl.ds(j * _CH, _CH)]],
                        rows_v.at[b], gsem[b])

                def wait_g(j, b):
                    pltpu.make_async_copy(
                        table_hbm.at[pos_l.at[pl.ds(j * _CH, _CH)]],
                        rows_v.at[b], gsem[b]).wait()

                def do_scat(j, b):
                    for g in range(_CH // 16):
                        loc2d[b, pl.ds(g * 16, 16)] = (
                            loc_l[pl.ds(j * _CH + g * 16, 16)])
                    pltpu.sync_copy(rows_v.at[b], acc.at[loc2d.at[b]],
                                    add=True)

                @pl.when(nch > 0)
                def _prime():
                    fire_g(0, 0)

                def pair_body(i, _):
                    j0 = 2 * i
                    j1 = j0 + 1

                    @pl.when(j1 < nch)
                    def _():
                        fire_g(j1, 1)

                    wait_g(j0, 0)
                    do_scat(j0, 0)

                    @pl.when(j1 < nch)
                    def _():
                        @pl.when(j1 + 1 < nch)
                        def _():
                            fire_g(j1 + 1, 0)

                        wait_g(j1, 1)
                        do_scat(j1, 1)

                    return 0

                lax.fori_loop(0, (nch + 1) // 2, pair_body, 0)
                plsc.subcore_barrier()

                for zc in range(NZCH):
                    pltpu.sync_copy(
                        acc.at[pl.ds(tid * R16 + zc * 128, 128)],
                        out_hbm.at[pl.ds(lo + tid * R16 + zc * 128, 128)])
                plsc.subcore_barrier()

    out = k(table, pos, seg, zeros_blk)
    return out[:num_segments]


def _seg_geom(num_segments):
    R = min(_SEG_R, ((num_segments + 4095) // 4096) * 2048)
    NR = (num_segments + R - 1) // R
    return R, NR, NR * R, (NR + 1) // 2


def _sc_segsum_plan(seg, num_segments, pos0, want_iota):
    K = seg.shape[0]
    R, NR, S_pad, KMAX = _seg_geom(num_segments)
    PS = K // 16
    NG = PS // 16
    STRIDE = PS + 144
    have0 = pos0 is not None
    nlists = (1 if have0 else 0) + (1 if want_iota else 0)

    mesh = plsc.VectorSubcoreMesh(core_axis_name="c", subcore_axis_name="s")
    lst_t = jax.ShapeDtypeStruct((NR * 16 * STRIDE,), jnp.int32)
    outs = [lst_t] * nlists + [lst_t, jax.ShapeDtypeStruct((1024,),
                                                           jnp.int32)]
    scr = [pltpu.VMEM((PS,), jnp.int32)]
    if have0:
        scr.append(pltpu.VMEM((PS,), jnp.int32))
    scr += [pltpu.VMEM((STRIDE,), jnp.int32) for _ in range(nlists + 1)]
    scr += [pltpu.VMEM((64,), jnp.int32),
            pltpu.SemaphoreType.DMA]

    @functools.partial(
        pl.kernel, mesh=mesh,
        compiler_params=pltpu.CompilerParams(needs_layout_passes=False),
        out_type=tuple(outs),
        scratch_types=scr,
    )
    def k(*refs):
        i = 0
        seg_hbm = refs[i]; i += 1
        pos0_hbm = None
        if have0:
            pos0_hbm = refs[i]; i += 1
        out_lists = refs[i:i + nlists + 1]; i += nlists + 1
        counts_hbm = refs[i]; i += 1
        seg_v = refs[i]; i += 1
        pos0_v = None
        if have0:
            pos0_v = refs[i]; i += 1
        lbufs = refs[i:i + nlists + 1]; i += nlists + 1
        counts_v = refs[i]; i += 1
        semw = refs[i]

        cid = lax.axis_index("c")
        tid = lax.axis_index("s")
        ebase = tid * PS
        pltpu.sync_copy(seg_hbm.at[pl.ds(ebase, PS)], seg_v)
        if have0:
            pltpu.sync_copy(pos0_hbm.at[pl.ds(ebase, PS)], pos0_v)
        lane = lax.iota(jnp.int32, 16)
        trash = jnp.full((16,), PS + 128, jnp.int32) + lane

        for kk in range(KMAX):
            rid = kk * 2 + cid

            @pl.when(rid < NR)
            def _range():
                lo = rid * R
                if kk > 0:
                    for b in lbufs:
                        pltpu.make_async_copy(
                            b, out_lists[0].at[pl.ds(0, STRIDE)],
                            semw).wait()

                def scan_body(g, cnt_vec):
                    sg = seg_v[pl.ds(g * 16, 16)]
                    m = (sg >= lo) & (sg < lo + R)
                    pref = plsc.cumsum(m.astype(jnp.int32))
                    offs = jnp.where(m, cnt_vec + pref - 1, trash)
                    j = 0
                    if have0:
                        plsc.store_scatter(lbufs[j], [offs],
                                           pos0_v[pl.ds(g * 16, 16)])
                        j += 1
                    if want_iota:
                        plsc.store_scatter(lbufs[j], [offs],
                                           ebase + g * 16 + lane)
                        j += 1
                    plsc.store_scatter(lbufs[j], [offs], sg - lo)
                    return cnt_vec + plsc.all_reduce_population_count(m)

                cnt_vec = lax.fori_loop(0, NG, scan_body,
                                        jnp.zeros((16,), jnp.int32))
                for g in range(8):
                    pad_off = cnt_vec + g * 16 + lane
                    for j in range(nlists):
                        plsc.store_scatter(lbufs[j], [pad_off],
                                           jnp.zeros((16,), jnp.int32))
                    plsc.store_scatter(lbufs[nlists], [pad_off],
                                       jnp.full((16,), R, jnp.int32))
                cnt = jnp.max(cnt_vec, axis=0)
                nch = (cnt + 127) // 128
                coffs = jnp.where(lane == 0, kk, 48 + lane)
                plsc.store_scatter(counts_v, [coffs],
                                   jnp.full((16,), 1, jnp.int32) * nch)
                lbase = (rid * 16 + tid) * STRIDE
                for j in range(nlists + 1):
                    pltpu.async_copy(lbufs[j],
                                     out_lists[j].at[pl.ds(lbase, STRIDE)],
                                     semw)

        for b in lbufs:
            pltpu.make_async_copy(b, out_lists[0].at[pl.ds(0, STRIDE)],
                                  semw).wait()
        wid = tid * _NC + cid
        pltpu.sync_copy(counts_v.at[pl.ds(0, 32)],
                        counts_hbm.at[pl.ds(wid * 32, 32)])

    args = (seg,) + ((pos0,) if have0 else ())
    return k(*args)


def _sc_segsum_consume(table, lists_pos, lists_loc, counts, num_segments):
    D = table.shape[1]
    R, NR, S_pad, KMAX = _seg_geom(num_segments)
    K16 = lists_pos.shape[0] // (NR * 16)
    STRIDE = K16
    R16 = R // 16
    NZCH = R16 // 128
    NCHMAX = STRIDE // 128

    zeros_blk = jnp.zeros((128, _D), jnp.float32)
    mesh = plsc.VectorSubcoreMesh(core_axis_name="c", subcore_axis_name="s")

    @functools.partial(
        pl.kernel, mesh=mesh,
        compiler_params=pltpu.CompilerParams(needs_layout_passes=False),
        out_type=jax.ShapeDtypeStruct((S_pad, D), jnp.float32),
        scratch_types=[
            pltpu.VMEM((STRIDE,), jnp.int32),
            pltpu.VMEM((STRIDE,), jnp.int32),
            pltpu.VMEM((2, 128), jnp.int32),
            pltpu.VMEM((2, 128, D), jnp.float32),
            pltpu.VMEM((64,), jnp.int32),
            pltpu.VMEM_SHARED((_SEG_R + 8, _D), jnp.float32),
            pltpu.SemaphoreType.DMA,
            pltpu.SemaphoreType.DMA,
            pltpu.SemaphoreType.DMA,
        ],
    )
    def k(table_hbm, lpos_hbm, lloc_hbm, counts_hbm, zeros_hbm, out_hbm,
          pos_l, loc_l, loc2d, rows_v, counts_v, acc, sem0, sem1, semz):
        cid = lax.axis_index("c")
        tid = lax.axis_index("s")
        wid = tid * _NC + cid
        pltpu.sync_copy(counts_hbm.at[pl.ds(wid * 32, 32)],
                        counts_v.at[pl.ds(0, 32)])
        lane = lax.iota(jnp.int32, 16)
        gsem = (sem0, sem1)

        for kk in range(KMAX):
            rid = kk * 2 + cid

            @pl.when(rid < NR)
            def _range():
                lo = rid * R
                zh = [pltpu.async_copy(
                    zeros_hbm, acc.at[pl.ds(tid * R16 + zc * 128, 128)],
                    semz) for zc in range(NZCH)]
                lbase = (rid * 16 + tid) * STRIDE
                pltpu.sync_copy(lpos_hbm.at[pl.ds(lbase, STRIDE)], pos_l)
                pltpu.sync_copy(lloc_hbm.at[pl.ds(lbase, STRIDE)], loc_l)
                cv = counts_v[pl.ds((kk // 16) * 16, 16)]
                nch = jnp.max(jnp.where(lane == (kk % 16), cv, 0), axis=0)
                for h in zh:
                    h.wait()
                plsc.subcore_barrier()

                def fire_g(j, b):
                    pltpu.async_copy(
                        table_hbm.at[pos_l.at[pl.ds(j * 128, 128)]],
                        rows_v.at[b], gsem[b])

                def wait_g(j, b):
                    pltpu.make_async_copy(
                        table_hbm.at[pos_l.at[pl.ds(j * 128, 128)]],
                        rows_v.at[b], gsem[b]).wait()

                def do_scat(j, b):
                    for g in range(8):
                        loc2d[b, pl.ds(g * 16, 16)] = (
                            loc_l[pl.ds(j * 128 + g * 16, 16)])
                    pltpu.sync_copy(rows_v.at[b], acc.at[loc2d.at[b]],
                                    add=True)

                @pl.when(nch > 0)
                def _prime():
                    fire_g(0, 0)

                def pair_body(i, _):
                    j0 = 2 * i
                    j1 = j0 + 1

                    @pl.when(j1 < nch)
                    def _():
                        fire_g(j1, 1)

                    wait_g(j0, 0)
                    do_scat(j0, 0)

                    @pl.when(j1 < nch)
                    def _():
                        @pl.when(j1 + 1 < nch)
                        def _():
                            fire_g(j1 + 1, 0)

                        wait_g(j1, 1)
                        do_scat(j1, 1)

                    return 0

                lax.fori_loop(0, (nch + 1) // 2, pair_body, 0)
                plsc.subcore_barrier()
                for zc in range(NZCH):
                    pltpu.sync_copy(
                        acc.at[pl.ds(tid * R16 + zc * 128, 128)],
                        out_hbm.at[pl.ds(lo + tid * R16 + zc * 128, 128)])
                plsc.subcore_barrier()

    out = k(table, lists_pos, lists_loc, counts, zeros_blk)
    return out[:num_segments]


def _iota(n):
    return jnp.arange(n, dtype=jnp.int32)


def kernel(node, connect, bond, bond_neighbour, W_node_w, W_node_b,
           W_node_final_w, W_node_final_b, W_bond_w, W_bond_b,
           W_bond_final_w, W_bond_final_b, W_z_w, W_z_b, W_r_w, W_r_b,
           U_w, W_w, W_b, W_n_w, W_n_b, U_n_w):
    i_idx = connect[0]
    j_idx = connect[1]
    ij_idx = bond_neighbour[0]
    ki_idx = bond_neighbour[1]
    N = node.shape[0]
    E = bond.shape[0]
    FN = node.shape[1]
    FB = bond.shape[1]

    nodei = _gather_rows(node, i_idx)

    mess_bond = _mm_fused([nodei, bond], [W_bond_w[:FN], W_bond_w[FN:]],
                          W_bond_b, "hswish")
    mess_node = _mm_fused([node], [W_node_w], W_node_b, "hswish",
                          block_rows=2000)
    pre_z = _mm_fused([nodei, bond], [W_z_w[:FN], W_z_w[FN:FN + FB]],
                      W_z_b, "none")
    pre_m = _mm_fused([nodei, bond], [W_w[:FN], W_w[FN:]], W_b, "none")
    pre_n = _mm_fused([node], [W_n_w], W_n_b, "none", block_rows=2000)

    pre_r_e = _mm_fused([nodei, bond], [W_r_w[:FN], W_r_w[FN:FN + FB]],
                        W_r_b, "none")
    pre_r = _gather_rows(pre_r_e, ij_idx)

    wz2 = W_z_w[FN + FB:]
    wr2 = W_r_w[FN + FB:]
    un1 = U_n_w[:_D]
    un2 = U_n_w[_D:]

    ij_ki, ij_io, ij_loc, ij_cnt = _sc_segsum_plan(ij_idx, E, ki_idx, True)
    j_io, j_loc, j_cnt = _sc_segsum_plan(j_idx, N, None, True)
    for _ in range(_LAYER):
        s_ij = _sc_segsum_consume(mess_bond, ij_ki, ij_loc, ij_cnt, E)
        mess_ki = _gather_rows(mess_bond, ki_idx)
        rmk = _rki_fused(pre_r, mess_ki, wr2)
        r_ij = _sc_segsum_consume(rmk, ij_io, ij_loc, ij_cnt, E)
        mess_bond = _bond_upd(pre_z, pre_m, s_ij, r_ij, wz2, U_w)
        aggr_node = _sc_segsum_consume(mess_bond, j_io, j_loc, j_cnt, N)
        mess_node = _node_upd(pre_n, mess_node, aggr_node, un1, un2)

    out_bond = _mm_fused([nodei, bond, mess_bond],
                         [W_bond_final_w[:FN], W_bond_final_w[FN:FN + FB],
                          W_bond_final_w[FN + FB:]],
                         W_bond_final_b, "hswish")
    out_node = _mm_fused([node, mess_node],
                         [W_node_final_w[:FN], W_node_final_w[FN:]],
                         W_node_final_b, "hswish", block_rows=2000)
    return (out_node, out_bond)


# speedup vs baseline: 1.3975x; 1.3975x over previous
import functools

import jax
import jax.numpy as jnp
from jax import lax
from jax.experimental import pallas as pl
from jax.experimental.pallas import tpu as pltpu
from jax.experimental.pallas import tpu_sc as plsc

_LAYER = 3
_D = 128
_NC, _NS = 2, 16
_NW = _NC * _NS


def _hswish(x):
    return x * jnp.clip(x + 3.0, 0.0, 6.0) / 6.0


def _mm_body(act, nx, *refs):
    in_refs = refs[:nx]
    w_refs = refs[nx:2 * nx]
    b_ref = refs[2 * nx]
    o_ref = refs[2 * nx + 1]
    acc = b_ref[...].astype(jnp.float32)
    for x_ref, w_ref in zip(in_refs, w_refs):
        acc = acc + jnp.dot(x_ref[...], w_ref[...],
                            preferred_element_type=jnp.float32)
    if act == "hswish":
        acc = _hswish(acc)
    elif act == "sigmoid":
        acc = jax.nn.sigmoid(acc)
    o_ref[...] = acc


def _mm_fused(xs, ws, bias, act, block_rows=2000):
    rows = xs[0].shape[0]
    grid = (rows // block_rows,)
    nx = len(xs)
    in_specs = (
        [pl.BlockSpec((block_rows, x.shape[1]), lambda i: (i, 0)) for x in xs]
        + [pl.BlockSpec(w.shape, lambda i: (0, 0)) for w in ws]
        + [pl.BlockSpec((1, _D), lambda i: (0, 0))]
    )
    return pl.pallas_call(
        functools.partial(_mm_body, act, nx),
        grid=grid,
        in_specs=in_specs,
        out_specs=pl.BlockSpec((block_rows, _D), lambda i: (i, 0)),
        out_shape=jax.ShapeDtypeStruct((rows, _D), jnp.float32),
    )(*xs, *ws, bias.reshape(1, _D))


def _rki_body(pre_ref, mk_ref, w_ref, o_ref):
    mk = mk_ref[...]
    r = jax.nn.sigmoid(pre_ref[...] + jnp.dot(mk, w_ref[...],
                                              preferred_element_type=jnp.float32))
    o_ref[...] = r * mk


def _rki_fused(pre, mk, w, block_rows=2000):
    rows = pre.shape[0]
    return pl.pallas_call(
        _rki_body,
        grid=(rows // block_rows,),
        in_specs=[
            pl.BlockSpec((block_rows, _D), lambda i: (i, 0)),
            pl.BlockSpec((block_rows, _D), lambda i: (i, 0)),
            pl.BlockSpec((_D, _D), lambda i: (0, 0)),
        ],
        out_specs=pl.BlockSpec((block_rows, _D), lambda i: (i, 0)),
        out_shape=jax.ShapeDtypeStruct((rows, _D), jnp.float32),
    )(pre, mk, w)


def _bond_upd_body(pre_z_ref, pre_m_ref, s_ref, r_ref, wz_ref, uw_ref, o_ref):
    s = s_ref[...]
    z = jax.nn.sigmoid(pre_z_ref[...] + jnp.dot(s, wz_ref[...],
                                                preferred_element_type=jnp.float32))
    m = jnp.tanh(pre_m_ref[...] + jnp.dot(r_ref[...], uw_ref[...],
                                          preferred_element_type=jnp.float32))
    o_ref[...] = (1.0 - z) * s + z * m


def _bond_upd(pre_z, pre_m, s, r, wz, uw, block_rows=2000):
    rows = pre_z.shape[0]
    bs = lambda: pl.BlockSpec((block_rows, _D), lambda i: (i, 0))
    return pl.pallas_call(
        _bond_upd_body,
        grid=(rows // block_rows,),
        in_specs=[bs(), bs(), bs(), bs(),
                  pl.BlockSpec((_D, _D), lambda i: (0, 0)),
                  pl.BlockSpec((_D, _D), lambda i: (0, 0))],
        out_specs=bs(),
        out_shape=jax.ShapeDtypeStruct((rows, _D), jnp.float32),
    )(pre_z, pre_m, s, r, wz, uw)


def _node_upd_body(pre_ref, mn_ref, ag_ref, u1_ref, u2_ref, o_ref):
    acc = pre_ref[...]
    acc = acc + jnp.dot(mn_ref[...], u1_ref[...], preferred_element_type=jnp.float32)
    acc = acc + jnp.dot(ag_ref[...], u2_ref[...], preferred_element_type=jnp.float32)
    o_ref[...] = _hswish(acc)


def _node_upd(pre_n, mn, aggr, u1, u2, block_rows=2000):
    rows = pre_n.shape[0]
    bs = lambda: pl.BlockSpec((block_rows, _D), lambda i: (i, 0))
    return pl.pallas_call(
        _node_upd_body,
        grid=(rows // block_rows,),
        in_specs=[bs(), bs(), bs(),
                  pl.BlockSpec((_D, _D), lambda i: (0, 0)),
                  pl.BlockSpec((_D, _D), lambda i: (0, 0))],
        out_specs=bs(),
        out_shape=jax.ShapeDtypeStruct((rows, _D), jnp.float32),
    )(pre_n, mn, aggr, u1, u2)


def _sc_gather(table, idx):
    K = idx.shape[0]
    D = table.shape[1]
    PW = K // _NW
    assert K % _NW == 0 and PW % 8 == 0, (K, PW)
    CH = min(128, PW)
    NFULL = PW // CH
    TAIL = PW - NFULL * CH
    assert TAIL % 8 == 0

    mesh = plsc.VectorSubcoreMesh(core_axis_name="c", subcore_axis_name="s")

    @functools.partial(
        pl.kernel, mesh=mesh,
        out_type=jax.ShapeDtypeStruct((K, D), jnp.float32),
        scratch_types=[
            pltpu.VMEM((PW,), jnp.int32),
            pltpu.VMEM((2, CH, D), jnp.float32),
            pltpu.SemaphoreType.DMA,
            pltpu.SemaphoreType.DMA,
        ],
    )
    def k(table_hbm, idx_hbm, out_hbm, idx_v, rows_v, sem0, sem1):
        wid = lax.axis_index("s") * _NC + lax.axis_index("c")
        base = wid * PW
        pltpu.sync_copy(idx_hbm.at[pl.ds(base, PW)], idx_v)
        sems = (sem0, sem1)

        def fire(c, b):
            pltpu.async_copy(table_hbm.at[idx_v.at[pl.ds(c * CH, CH)]],
                             rows_v.at[b], sems[b])

        def wait_write(c, b):
            pltpu.make_async_copy(
                table_hbm.at[idx_v.at[pl.ds(c * CH, CH)]],
                rows_v.at[b], sems[b]).wait()
            pltpu.sync_copy(rows_v.at[b],
                            out_hbm.at[pl.ds(base + c * CH, CH)])

        fire(0, 0)
        for c in range(1, NFULL):
            fire(c, c & 1)
            wait_write(c - 1, (c - 1) & 1)
        wait_write(NFULL - 1, (NFULL - 1) & 1)
        if TAIL:
            pltpu.async_copy(
                table_hbm.at[idx_v.at[pl.ds(NFULL * CH, TAIL)]],
                rows_v.at[1, pl.ds(0, TAIL)], sem1).wait()
            pltpu.sync_copy(rows_v.at[1, pl.ds(0, TAIL)],
                            out_hbm.at[pl.ds(base + NFULL * CH, TAIL)])

    return k(table, idx)


def _gather_rows(table, idx):
    return _sc_gather(table, idx)


_SEG_R = 8192
_CH = 64


def _sc_segsum_gather(table, pos, seg, num_segments):
    K = seg.shape[0]
    D = table.shape[1]
    assert D == _D
    R = min(_SEG_R, ((num_segments + 4095) // 4096) * 2048)
    NR = (num_segments + R - 1) // R
    S_pad = NR * R
    R16 = R // 16
    NZCH = R16 // 128
    assert R16 % 128 == 0
    PS = K // 16
    NG = PS // 16
    assert K % 256 == 0
    LCAP = PS + 144
    KMAX = (NR + 1) // 2

    zeros_blk = jnp.zeros((128, _D), jnp.float32)
    mesh = plsc.VectorSubcoreMesh(core_axis_name="c", subcore_axis_name="s")

    @functools.partial(
        pl.kernel, mesh=mesh,
        compiler_params=pltpu.CompilerParams(needs_layout_passes=False),
        out_type=jax.ShapeDtypeStruct((S_pad, D), jnp.float32),
        scratch_types=[
            pltpu.VMEM((PS,), jnp.int32),
            pltpu.VMEM((PS,), jnp.int32),
            pltpu.VMEM((LCAP,), jnp.int32),
            pltpu.VMEM((LCAP,), jnp.int32),
            pltpu.VMEM((2, _CH), jnp.int32),
            pltpu.VMEM((2, _CH, D), jnp.float32),
            pltpu.VMEM_SHARED((_SEG_R + 8, _D), jnp.float32),
            pltpu.SemaphoreType.DMA,
            pltpu.SemaphoreType.DMA,
            pltpu.SemaphoreType.DMA,
            pltpu.SemaphoreType.DMA,
        ],
    )
    def k(table_hbm, pos_hbm, seg_hbm, zeros_hbm, out_hbm,
          seg_v, pos_v, pos_l, loc_l, loc2d, rows_v, acc,
          sem0, sem1, semz, semw):
        cid = lax.axis_index("c")
        tid = lax.axis_index("s")
        ebase = tid * PS
        pltpu.sync_copy(seg_hbm.at[pl.ds(ebase, PS)], seg_v)
        pltpu.sync_copy(pos_hbm.at[pl.ds(ebase, PS)], pos_v)
        gsem = (sem0, sem1)

        def drain_writeout():
            for zc in range(NZCH):
                pltpu.make_async_copy(
                    zeros_hbm, acc.at[pl.ds(tid * R16 + zc * 128, 128)],
                    semw).wait()

        for kk in range(KMAX):
            rid = kk * 2 + cid

            @pl.when(rid < NR)
            def _range():
                lo = rid * R

                lane = lax.iota(jnp.int32, 16)
                trash = jnp.full((16,), PS + 128, jnp.int32) + lane

                def scan_body(g, cnt_vec):
                    sg = seg_v[pl.ds(g * 16, 16)]
                    m = (sg >= lo) & (sg < lo + R)
                    pref = plsc.cumsum(m.astype(jnp.int32))
                    offs = jnp.where(m, cnt_vec + pref - 1, trash)
                    plsc.store_scatter(pos_l, [offs],
                                       pos_v[pl.ds(g * 16, 16)])
                    plsc.store_scatter(loc_l, [offs], sg - lo)
                    return cnt_vec + plsc.all_reduce_population_count(m)

                cnt_vec = lax.fori_loop(0, NG, scan_body,
                                        jnp.zeros((16,), jnp.int32))
                for g in range(_CH // 16):
                    pad_off = cnt_vec + g * 16 + lane
                    plsc.store_scatter(pos_l, [pad_off],
                                       jnp.zeros((16,), jnp.int32))
                    plsc.store_scatter(loc_l, [pad_off],
                                       jnp.full((16,), R, jnp.int32))
                cnt = jnp.max(cnt_vec, axis=0)
                nch = (cnt + _CH - 1) // _CH
                if kk > 0:
                    drain_writeout()
                zh = [pltpu.async_copy(
                    zeros_hbm, acc.at[pl.ds(tid * R16 + zc * 128, 128)],
                    semz) for zc in range(NZCH)]
                for h in zh:
                    h.wait()
                plsc.subcore_barrier()

                def fire_g(j, b):
                    pltpu.async_copy(
                        table_hbm.at[pos_l.at[pl.ds(j * _CH, _CH)]],
                        rows_v.at[b], gsem[b])

                def wait_g(j, b):
                    pltpu.make_async_copy(
                        table_hbm.at[pos_l.at[pl.ds(j * _CH, _CH)]],
                        rows_v.at[b], gsem[b]).wait()

                def do_scat(j, b):
                    for g in range(_CH // 16):
                        loc2d[b, pl.ds(g * 16, 16)] = (
                            loc_l[pl.ds(j * _CH + g * 16, 16)])
                    pltpu.sync_copy(rows_v.at[b], acc.at[loc2d.at[b]],
                                    add=True)

                @pl.when(nch > 0)
                def _prime():
                    fire_g(0, 0)

                def pair_body(i, _):
                    j0 = 2 * i
                    j1 = j0 + 1

                    @pl.when(j1 < nch)
                    def _():
                        fire_g(j1, 1)

                    wait_g(j0, 0)
                    do_scat(j0, 0)

                    @pl.when(j1 < nch)
                    def _():
                        @pl.when(j1 + 1 < nch)
                        def _():
                            fire_g(j1 + 1, 0)

                        wait_g(j1, 1)
                        do_scat(j1, 1)

                    return 0

                lax.fori_loop(0, (nch + 1) // 2, pair_body, 0)
                plsc.subcore_barrier()

                for zc in range(NZCH):
                    pltpu.async_copy(
                        acc.at[pl.ds(tid * R16 + zc * 128, 128)],
                        out_hbm.at[pl.ds(lo + tid * R16 + zc * 128, 128)],
                        semw)

        drain_writeout()

    out = k(table, pos, seg, zeros_blk)
    return out[:num_segments]


def _seg_geom(num_segments):
    R = min(_SEG_R, ((num_segments + 4095) // 4096) * 2048)
    NR = (num_segments + R - 1) // R
    return R, NR, NR * R, (NR + 1) // 2


def _sc_segsum_plan(seg, num_segments, pos0, want_iota):
    K = seg.shape[0]
    R, NR, S_pad, KMAX = _seg_geom(num_segments)
    PS = K // 16
    NG = PS // 16
    STRIDE = PS + 144
    have0 = pos0 is not None
    nlists = (1 if have0 else 0) + (1 if want_iota else 0)

    mesh = plsc.VectorSubcoreMesh(core_axis_name="c", subcore_axis_name="s")
    lst_t = jax.ShapeDtypeStruct((NR * 16 * STRIDE,), jnp.int32)
    outs = [lst_t] * nlists + [lst_t, jax.ShapeDtypeStruct((1024,),
                                                           jnp.int32)]
    scr = [pltpu.VMEM((PS,), jnp.int32)]
    if have0:
        scr.append(pltpu.VMEM((PS,), jnp.int32))
    scr += [pltpu.VMEM((STRIDE,), jnp.int32) for _ in range(nlists + 1)]
    scr += [pltpu.VMEM((64,), jnp.int32),
            pltpu.SemaphoreType.DMA]

    @functools.partial(
        pl.kernel, mesh=mesh,
        compiler_params=pltpu.CompilerParams(needs_layout_passes=False),
        out_type=tuple(outs),
        scratch_types=scr,
    )
    def k(*refs):
        i = 0
        seg_hbm = refs[i]; i += 1
        pos0_hbm = None
        if have0:
            pos0_hbm = refs[i]; i += 1
        out_lists = refs[i:i + nlists + 1]; i += nlists + 1
        counts_hbm = refs[i]; i += 1
        seg_v = refs[i]; i += 1
        pos0_v = None
        if have0:
            pos0_v = refs[i]; i += 1
        lbufs = refs[i:i + nlists + 1]; i += nlists + 1
        counts_v = refs[i]; i += 1
        semw = refs[i]

        cid = lax.axis_index("c")
        tid = lax.axis_index("s")
        ebase = tid * PS
        pltpu.sync_copy(seg_hbm.at[pl.ds(ebase, PS)], seg_v)
        if have0:
            pltpu.sync_copy(pos0_hbm.at[pl.ds(ebase, PS)], pos0_v)
        lane = lax.iota(jnp.int32, 16)
        trash = jnp.full((16,), PS + 128, jnp.int32) + lane

        for kk in range(KMAX):
            rid = kk * 2 + cid

            @pl.when(rid < NR)
            def _range():
                lo = rid * R
                if kk > 0:
                    for b in lbufs:
                        pltpu.make_async_copy(
                            b, out_lists[0].at[pl.ds(0, STRIDE)],
                            semw).wait()

                def scan_body(g, cnt_vec):
                    sg = seg_v[pl.ds(g * 16, 16)]
                    m = (sg >= lo) & (sg < lo + R)
                    pref = plsc.cumsum(m.astype(jnp.int32))
                    offs = jnp.where(m, cnt_vec + pref - 1, trash)
                    j = 0
                    if have0:
                        plsc.store_scatter(lbufs[j], [offs],
                                           pos0_v[pl.ds(g * 16, 16)])
                        j += 1
                    if want_iota:
                        plsc.store_scatter(lbufs[j], [offs],
                                           ebase + g * 16 + lane)
                        j += 1
                    plsc.store_scatter(lbufs[j], [offs], sg - lo)
                    return cnt_vec + plsc.all_reduce_population_count(m)

                cnt_vec = lax.fori_loop(0, NG, scan_body,
                                        jnp.zeros((16,), jnp.int32))
                for g in range(8):
                    pad_off = cnt_vec + g * 16 + lane
                    for j in range(nlists):
                        plsc.store_scatter(lbufs[j], [pad_off],
                                           jnp.zeros((16,), jnp.int32))
                    plsc.store_scatter(lbufs[nlists], [pad_off],
                                       jnp.full((16,), R, jnp.int32))
                cnt = jnp.max(cnt_vec, axis=0)
                nch = (cnt + 127) // 128
                coffs = jnp.where(lane == 0, kk, 48 + lane)
                plsc.store_scatter(counts_v, [coffs],
                                   jnp.full((16,), 1, jnp.int32) * nch)
                lbase = (rid * 16 + tid) * STRIDE
                for j in range(nlists + 1):
                    pltpu.async_copy(lbufs[j],
                                     out_lists[j].at[pl.ds(lbase, STRIDE)],
                                     semw)

        for b in lbufs:
            pltpu.make_async_copy(b, out_lists[0].at[pl.ds(0, STRIDE)],
                                  semw).wait()
        wid = tid * _NC + cid
        pltpu.sync_copy(counts_v.at[pl.ds(0, 32)],
                        counts_hbm.at[pl.ds(wid * 32, 32)])

    args = (seg,) + ((pos0,) if have0 else ())
    return k(*args)


def _sc_segsum_consume(table, lists_pos, lists_loc, counts, num_segments):
    D = table.shape[1]
    R, NR, S_pad, KMAX = _seg_geom(num_segments)
    K16 = lists_pos.shape[0] // (NR * 16)
    STRIDE = K16
    R16 = R // 16
    NZCH = R16 // 128
    NCHMAX = STRIDE // 128

    zeros_blk = jnp.zeros((128, _D), jnp.float32)
    mesh = plsc.VectorSubcoreMesh(core_axis_name="c", subcore_axis_name="s")

    @functools.partial(
        pl.kernel, mesh=mesh,
        compiler_params=pltpu.CompilerParams(needs_layout_passes=False),
        out_type=jax.ShapeDtypeStruct((S_pad, D), jnp.float32),
        scratch_types=[
            pltpu.VMEM((STRIDE,), jnp.int32),
            pltpu.VMEM((STRIDE,), jnp.int32),
            pltpu.VMEM((2, 128), jnp.int32),
            pltpu.VMEM((2, 128, D), jnp.float32),
            pltpu.VMEM((64,), jnp.int32),
            pltpu.VMEM_SHARED((_SEG_R + 8, _D), jnp.float32),
            pltpu.SemaphoreType.DMA,
            pltpu.SemaphoreType.DMA,
            pltpu.SemaphoreType.DMA,
        ],
    )
    def k(table_hbm, lpos_hbm, lloc_hbm, counts_hbm, zeros_hbm, out_hbm,
          pos_l, loc_l, loc2d, rows_v, counts_v, acc, sem0, sem1, semz):
        cid = lax.axis_index("c")
        tid = lax.axis_index("s")
        wid = tid * _NC + cid
        pltpu.sync_copy(counts_hbm.at[pl.ds(wid * 32, 32)],
                        counts_v.at[pl.ds(0, 32)])
        lane = lax.iota(jnp.int32, 16)
        gsem = (sem0, sem1)

        for kk in range(KMAX):
            rid = kk * 2 + cid

            @pl.when(rid < NR)
            def _range():
                lo = rid * R
                zh = [pltpu.async_copy(
                    zeros_hbm, acc.at[pl.ds(tid * R16 + zc * 128, 128)],
                    semz) for zc in range(NZCH)]
                lbase = (rid * 16 + tid) * STRIDE
                pltpu.sync_copy(lpos_hbm.at[pl.ds(lbase, STRIDE)], pos_l)
                pltpu.sync_copy(lloc_hbm.at[pl.ds(lbase, STRIDE)], loc_l)
                cv = counts_v[pl.ds((kk // 16) * 16, 16)]
                nch = jnp.max(jnp.where(lane == (kk % 16), cv, 0), axis=0)
                for h in zh:
                    h.wait()
                plsc.subcore_barrier()

                def fire_g(j, b):
                    pltpu.async_copy(
                        table_hbm.at[pos_l.at[pl.ds(j * 128, 128)]],
                        rows_v.at[b], gsem[b])

                def wait_g(j, b):
                    pltpu.make_async_copy(
                        table_hbm.at[pos_l.at[pl.ds(j * 128, 128)]],
                        rows_v.at[b], gsem[b]).wait()

                def do_scat(j, b):
                    for g in range(8):
                        loc2d[b, pl.ds(g * 16, 16)] = (
                            loc_l[pl.ds(j * 128 + g * 16, 16)])
                    pltpu.sync_copy(rows_v.at[b], acc.at[loc2d.at[b]],
                                    add=True)

                @pl.when(nch > 0)
                def _prime():
                    fire_g(0, 0)

                def pair_body(i, _):
                    j0 = 2 * i
                    j1 = j0 + 1

                    @pl.when(j1 < nch)
                    def _():
                        fire_g(j1, 1)

                    wait_g(j0, 0)
                    do_scat(j0, 0)

                    @pl.when(j1 < nch)
                    def _():
                        @pl.when(j1 + 1 < nch)
                        def _():
                            fire_g(j1 + 1, 0)

                        wait_g(j1, 1)
                        do_scat(j1, 1)

                    return 0

                lax.fori_loop(0, (nch + 1) // 2, pair_body, 0)
                plsc.subcore_barrier()
                for zc in range(NZCH):
                    pltpu.sync_copy(
                        acc.at[pl.ds(tid * R16 + zc * 128, 128)],
                        out_hbm.at[pl.ds(lo + tid * R16 + zc * 128, 128)])
                plsc.subcore_barrier()

    out = k(table, lists_pos, lists_loc, counts, zeros_blk)
    return out[:num_segments]


def _iota(n):
    return jnp.arange(n, dtype=jnp.int32)


def kernel(node, connect, bond, bond_neighbour, W_node_w, W_node_b,
           W_node_final_w, W_node_final_b, W_bond_w, W_bond_b,
           W_bond_final_w, W_bond_final_b, W_z_w, W_z_b, W_r_w, W_r_b,
           U_w, W_w, W_b, W_n_w, W_n_b, U_n_w):
    i_idx = connect[0]
    j_idx = connect[1]
    ij_idx = bond_neighbour[0]
    ki_idx = bond_neighbour[1]
    N = node.shape[0]
    E = bond.shape[0]
    FN = node.shape[1]
    FB = bond.shape[1]

    nodei = _gather_rows(node, i_idx)

    mess_bond = _mm_fused([nodei, bond], [W_bond_w[:FN], W_bond_w[FN:]],
                          W_bond_b, "hswish")
    mess_node = _mm_fused([node], [W_node_w], W_node_b, "hswish",
                          block_rows=2000)
    pre_z = _mm_fused([nodei, bond], [W_z_w[:FN], W_z_w[FN:FN + FB]],
                      W_z_b, "none")
    pre_m = _mm_fused([nodei, bond], [W_w[:FN], W_w[FN:]], W_b, "none")
    pre_n = _mm_fused([node], [W_n_w], W_n_b, "none", block_rows=2000)

    pre_r_e = _mm_fused([nodei, bond], [W_r_w[:FN], W_r_w[FN:FN + FB]],
                        W_r_b, "none")
    pre_r = _gather_rows(pre_r_e, ij_idx)

    wz2 = W_z_w[FN + FB:]
    wr2 = W_r_w[FN + FB:]
    un1 = U_n_w[:_D]
    un2 = U_n_w[_D:]

    iota_enb = _iota(ij_idx.shape[0])
    iota_e = _iota(E)
    for _ in range(_LAYER):
        s_ij = _sc_segsum_gather(mess_bond, ki_idx, ij_idx, E)
        mess_ki = _gather_rows(mess_bond, ki_idx)
        rmk = _rki_fused(pre_r, mess_ki, wr2)
        r_ij = _sc_segsum_gather(rmk, iota_enb, ij_idx, E)
        mess_bond = _bond_upd(pre_z, pre_m, s_ij, r_ij, wz2, U_w)
        aggr_node = _sc_segsum_gather(mess_bond, iota_e, j_idx, N)
        mess_node = _node_upd(pre_n, mess_node, aggr_node, un1, un2)

    out_bond = _mm_fused([nodei, bond, mess_bond],
                         [W_bond_final_w[:FN], W_bond_final_w[FN:FN + FB],
                          W_bond_final_w[FN + FB:]],
                         W_bond_final_b, "hswish")
    out_node = _mm_fused([node, mess_node],
                         [W_node_final_w[:FN], W_node_final_w[FN:]],
                         W_node_final_b, "hswish", block_rows=2000)
    return (out_node, out_bond)


# speedup vs baseline: 1.4388x; 1.0295x over previous
import functools

import jax
import jax.numpy as jnp
from jax import lax
from jax.experimental import pallas as pl
from jax.experimental.pallas import tpu as pltpu
from jax.experimental.pallas import tpu_sc as plsc

_LAYER = 3
_D = 128
_NC, _NS = 2, 16
_NW = _NC * _NS


def _hswish(x):
    return x * jnp.clip(x + 3.0, 0.0, 6.0) / 6.0


def _mm_body(act, nx, *refs):
    in_refs = refs[:nx]
    w_refs = refs[nx:2 * nx]
    b_ref = refs[2 * nx]
    o_ref = refs[2 * nx + 1]
    acc = b_ref[...].astype(jnp.float32)
    for x_ref, w_ref in zip(in_refs, w_refs):
        acc = acc + jnp.dot(x_ref[...], w_ref[...],
                            preferred_element_type=jnp.float32)
    if act == "hswish":
        acc = _hswish(acc)
    elif act == "sigmoid":
        acc = jax.nn.sigmoid(acc)
    o_ref[...] = acc


def _mm_fused(xs, ws, bias, act, block_rows=2000):
    rows = xs[0].shape[0]
    grid = (rows // block_rows,)
    nx = len(xs)
    in_specs = (
        [pl.BlockSpec((block_rows, x.shape[1]), lambda i: (i, 0)) for x in xs]
        + [pl.BlockSpec(w.shape, lambda i: (0, 0)) for w in ws]
        + [pl.BlockSpec((1, _D), lambda i: (0, 0))]
    )
    return pl.pallas_call(
        functools.partial(_mm_body, act, nx),
        grid=grid,
        in_specs=in_specs,
        out_specs=pl.BlockSpec((block_rows, _D), lambda i: (i, 0)),
        out_shape=jax.ShapeDtypeStruct((rows, _D), jnp.float32),
    )(*xs, *ws, bias.reshape(1, _D))


def _rki_body(pre_ref, mk_ref, w_ref, o_ref):
    mk = mk_ref[...]
    r = jax.nn.sigmoid(pre_ref[...] + jnp.dot(mk, w_ref[...],
                                              preferred_element_type=jnp.float32))
    o_ref[...] = r * mk


def _rki_fused(pre, mk, w, block_rows=2000):
    rows = pre.shape[0]
    return pl.pallas_call(
        _rki_body,
        grid=(rows // block_rows,),
        in_specs=[
            pl.BlockSpec((block_rows, _D), lambda i: (i, 0)),
            pl.BlockSpec((block_rows, _D), lambda i: (i, 0)),
            pl.BlockSpec((_D, _D), lambda i: (0, 0)),
        ],
        out_specs=pl.BlockSpec((block_rows, _D), lambda i: (i, 0)),
        out_shape=jax.ShapeDtypeStruct((rows, _D), jnp.float32),
    )(pre, mk, w)


def _bond_upd_body(pre_z_ref, pre_m_ref, s_ref, r_ref, wz_ref, uw_ref, o_ref):
    s = s_ref[...]
    z = jax.nn.sigmoid(pre_z_ref[...] + jnp.dot(s, wz_ref[...],
                                                preferred_element_type=jnp.float32))
    m = jnp.tanh(pre_m_ref[...] + jnp.dot(r_ref[...], uw_ref[...],
                                          preferred_element_type=jnp.float32))
    o_ref[...] = (1.0 - z) * s + z * m


def _bond_upd(pre_z, pre_m, s, r, wz, uw, block_rows=2000):
    rows = pre_z.shape[0]
    bs = lambda: pl.BlockSpec((block_rows, _D), lambda i: (i, 0))
    return pl.pallas_call(
        _bond_upd_body,
        grid=(rows // block_rows,),
        in_specs=[bs(), bs(), bs(), bs(),
                  pl.BlockSpec((_D, _D), lambda i: (0, 0)),
                  pl.BlockSpec((_D, _D), lambda i: (0, 0))],
        out_specs=bs(),
        out_shape=jax.ShapeDtypeStruct((rows, _D), jnp.float32),
    )(pre_z, pre_m, s, r, wz, uw)


def _node_upd_body(pre_ref, mn_ref, ag_ref, u1_ref, u2_ref, o_ref):
    acc = pre_ref[...]
    acc = acc + jnp.dot(mn_ref[...], u1_ref[...], preferred_element_type=jnp.float32)
    acc = acc + jnp.dot(ag_ref[...], u2_ref[...], preferred_element_type=jnp.float32)
    o_ref[...] = _hswish(acc)


def _node_upd(pre_n, mn, aggr, u1, u2, block_rows=2000):
    rows = pre_n.shape[0]
    bs = lambda: pl.BlockSpec((block_rows, _D), lambda i: (i, 0))
    return pl.pallas_call(
        _node_upd_body,
        grid=(rows // block_rows,),
        in_specs=[bs(), bs(), bs(),
                  pl.BlockSpec((_D, _D), lambda i: (0, 0)),
                  pl.BlockSpec((_D, _D), lambda i: (0, 0))],
        out_specs=bs(),
        out_shape=jax.ShapeDtypeStruct((rows, _D), jnp.float32),
    )(pre_n, mn, aggr, u1, u2)


def _sc_gather(table, idx):
    K = idx.shape[0]
    D = table.shape[1]
    PW = K // _NW
    assert K % _NW == 0 and PW % 8 == 0, (K, PW)
    CH = min(128, PW)
    NFULL = PW // CH
    TAIL = PW - NFULL * CH
    assert TAIL % 8 == 0

    mesh = plsc.VectorSubcoreMesh(core_axis_name="c", subcore_axis_name="s")

    @functools.partial(
        pl.kernel, mesh=mesh,
        out_type=jax.ShapeDtypeStruct((K, D), jnp.float32),
        scratch_types=[
            pltpu.VMEM((PW,), jnp.int32),
            pltpu.VMEM((2, CH, D), jnp.float32),
            pltpu.SemaphoreType.DMA,
            pltpu.SemaphoreType.DMA,
        ],
    )
    def k(table_hbm, idx_hbm, out_hbm, idx_v, rows_v, sem0, sem1):
        wid = lax.axis_index("s") * _NC + lax.axis_index("c")
        base = wid * PW
        pltpu.sync_copy(idx_hbm.at[pl.ds(base, PW)], idx_v)
        sems = (sem0, sem1)

        def fire(c, b):
            pltpu.async_copy(table_hbm.at[idx_v.at[pl.ds(c * CH, CH)]],
                             rows_v.at[b], sems[b])

        def wait_write(c, b):
            pltpu.make_async_copy(
                table_hbm.at[idx_v.at[pl.ds(c * CH, CH)]],
                rows_v.at[b], sems[b]).wait()
            pltpu.sync_copy(rows_v.at[b],
                            out_hbm.at[pl.ds(base + c * CH, CH)])

        fire(0, 0)
        for c in range(1, NFULL):
            fire(c, c & 1)
            wait_write(c - 1, (c - 1) & 1)
        wait_write(NFULL - 1, (NFULL - 1) & 1)
        if TAIL:
            pltpu.async_copy(
                table_hbm.at[idx_v.at[pl.ds(NFULL * CH, TAIL)]],
                rows_v.at[1, pl.ds(0, TAIL)], sem1).wait()
            pltpu.sync_copy(rows_v.at[1, pl.ds(0, TAIL)],
                            out_hbm.at[pl.ds(base + NFULL * CH, TAIL)])

    return k(table, idx)


def _gather_rows(table, idx):
    return _sc_gather(table, idx)


_SEG_R = 8192
_CH = 64


def _sc_segsum_gather(table, pos, seg, num_segments):
    K = seg.shape[0]
    D = table.shape[1]
    assert D == _D
    R = min(_SEG_R, ((num_segments + 4095) // 4096) * 2048)
    NR = (num_segments + R - 1) // R
    S_pad = NR * R
    R16 = R // 16
    NZCH = R16 // 128
    assert R16 % 128 == 0
    PS = K // 16
    NG = PS // 16
    assert K % 256 == 0
    LCAP = PS + 144
    KMAX = (NR + 1) // 2

    zeros_blk = jnp.zeros((128, _D), jnp.float32)
    mesh = plsc.VectorSubcoreMesh(core_axis_name="c", subcore_axis_name="s")

    @functools.partial(
        pl.kernel, mesh=mesh,
        compiler_params=pltpu.CompilerParams(needs_layout_passes=False),
        out_type=jax.ShapeDtypeStruct((S_pad, D), jnp.float32),
        scratch_types=[
            pltpu.VMEM((PS,), jnp.int32),
            pltpu.VMEM((PS,), jnp.int32),
            pltpu.VMEM((LCAP,), jnp.int32),
            pltpu.VMEM((LCAP,), jnp.int32),
            pltpu.VMEM((2, _CH), jnp.int32),
            pltpu.VMEM((2, _CH, D), jnp.float32),
            pltpu.VMEM_SHARED((_SEG_R + 8, _D), jnp.float32),
            pltpu.SemaphoreType.DMA,
            pltpu.SemaphoreType.DMA,
            pltpu.SemaphoreType.DMA,
        ],
    )
    def k(table_hbm, pos_hbm, seg_hbm, zeros_hbm, out_hbm,
          seg_v, pos_v, pos_l, loc_l, loc2d, rows_v, acc,
          sem0, sem1, semz):
        cid = lax.axis_index("c")
        tid = lax.axis_index("s")
        ebase = tid * PS
        pltpu.sync_copy(seg_hbm.at[pl.ds(ebase, PS)], seg_v)
        pltpu.sync_copy(pos_hbm.at[pl.ds(ebase, PS)], pos_v)
        gsem = (sem0, sem1)

        for kk in range(KMAX):
            rid = kk * 2 + cid

            @pl.when(rid < NR)
            def _range():
                lo = rid * R
                zh = [pltpu.async_copy(
                    zeros_hbm, acc.at[pl.ds(tid * R16 + zc * 128, 128)],
                    semz) for zc in range(NZCH)]

                lane = lax.iota(jnp.int32, 16)
                trash = jnp.full((16,), PS + 128, jnp.int32) + lane

                def scan_body(g, cnt_vec):
                    sg = seg_v[pl.ds(g * 16, 16)]
                    m = (sg >= lo) & (sg < lo + R)
                    pref = plsc.cumsum(m.astype(jnp.int32))
                    offs = jnp.where(m, cnt_vec + pref - 1, trash)
                    plsc.store_scatter(pos_l, [offs],
                                       pos_v[pl.ds(g * 16, 16)])
                    plsc.store_scatter(loc_l, [offs], sg - lo)
                    return cnt_vec + plsc.all_reduce_population_count(m)

                cnt_vec = lax.fori_loop(0, NG, scan_body,
                                        jnp.zeros((16,), jnp.int32))
                for g in range(_CH // 16):
                    pad_off = cnt_vec + g * 16 + lane
                    plsc.store_scatter(pos_l, [pad_off],
                                       jnp.zeros((16,), jnp.int32))
                    plsc.store_scatter(loc_l, [pad_off],
                                       jnp.full((16,), R, jnp.int32))
                cnt = jnp.max(cnt_vec, axis=0)
                nch = (cnt + _CH - 1) // _CH
                for h in zh:
                    h.wait()
                plsc.subcore_barrier()

                def fire_g(j, b):
                    pltpu.async_copy(
                        table_hbm.at[pos_l.at[pl.ds(j * _CH, _CH)]],
                        rows_v.at[b], gsem[b])

                def wait_g(j, b):
                    pltpu.make_async_copy(
                        table_hbm.at[pos_l.at[pl.ds(j * _CH, _CH)]],
                        rows_v.at[b], gsem[b]).wait()

                def do_scat(j, b):
                    for g in range(_CH // 16):
                        loc2d[b, pl.ds(g * 16, 16)] = (
                            loc_l[pl.ds(j * _CH + g * 16, 16)])
                    pltpu.sync_copy(rows_v.at[b], acc.at[loc2d.at[b]],
                                    add=True)

                @pl.when(nch > 0)
                def _prime():
                    fire_g(0, 0)

                def pair_body(i, _):
                    j0 = 2 * i
                    j1 = j0 + 1

                    @pl.when(j1 < nch)
                    def _():
                        fire_g(j1, 1)

                    wait_g(j0, 0)
                    do_scat(j0, 0)

                    @pl.when(j1 < nch)
                    def _():
                        @pl.when(j1 + 1 < nch)
                        def _():
                            fire_g(j1 + 1, 0)

                        wait_g(j1, 1)
                        do_scat(j1, 1)

                    return 0

                lax.fori_loop(0, (nch + 1) // 2, pair_body, 0)
                plsc.subcore_barrier()

                for zc in range(NZCH):
                    pltpu.sync_copy(
                        acc.at[pl.ds(tid * R16 + zc * 128, 128)],
                        out_hbm.at[pl.ds(lo + tid * R16 + zc * 128, 128)])

    out = k(table, pos, seg, zeros_blk)
    return out[:num_segments]


def _seg_geom(num_segments):
    R = min(_SEG_R, ((num_segments + 4095) // 4096) * 2048)
    NR = (num_segments + R - 1) // R
    return R, NR, NR * R, (NR + 1) // 2


def _sc_segsum_plan(seg, num_segments, pos0, want_iota):
    K = seg.shape[0]
    R, NR, S_pad, KMAX = _seg_geom(num_segments)
    PS = K // 16
    NG = PS // 16
    STRIDE = PS + 144
    have0 = pos0 is not None
    nlists = (1 if have0 else 0) + (1 if want_iota else 0)

    mesh = plsc.VectorSubcoreMesh(core_axis_name="c", subcore_axis_name="s")
    lst_t = jax.ShapeDtypeStruct((NR * 16 * STRIDE,), jnp.int32)
    outs = [lst_t] * nlists + [lst_t, jax.ShapeDtypeStruct((1024,),
                                                           jnp.int32)]
    scr = [pltpu.VMEM((PS,), jnp.int32)]
    if have0:
        scr.append(pltpu.VMEM((PS,), jnp.int32))
    scr += [pltpu.VMEM((STRIDE,), jnp.int32) for _ in range(nlists + 1)]
    scr += [pltpu.VMEM((64,), jnp.int32),
            pltpu.SemaphoreType.DMA]

    @functools.partial(
        pl.kernel, mesh=mesh,
        compiler_params=pltpu.CompilerParams(needs_layout_passes=False),
        out_type=tuple(outs),
        scratch_types=scr,
    )
    def k(*refs):
        i = 0
        seg_hbm = refs[i]; i += 1
        pos0_hbm = None
        if have0:
            pos0_hbm = refs[i]; i += 1
        out_lists = refs[i:i + nlists + 1]; i += nlists + 1
        counts_hbm = refs[i]; i += 1
        seg_v = refs[i]; i += 1
        pos0_v = None
        if have0:
            pos0_v = refs[i]; i += 1
        lbufs = refs[i:i + nlists + 1]; i += nlists + 1
        counts_v = refs[i]; i += 1
        semw = refs[i]

        cid = lax.axis_index("c")
        tid = lax.axis_index("s")
        ebase = tid * PS
        pltpu.sync_copy(seg_hbm.at[pl.ds(ebase, PS)], seg_v)
        if have0:
            pltpu.sync_copy(pos0_hbm.at[pl.ds(ebase, PS)], pos0_v)
        lane = lax.iota(jnp.int32, 16)
        trash = jnp.full((16,), PS + 128, jnp.int32) + lane

        for kk in range(KMAX):
            rid = kk * 2 + cid

            @pl.when(rid < NR)
            def _range():
                lo = rid * R
                if kk > 0:
                    for b in lbufs:
                        pltpu.make_async_copy(
                            b, out_lists[0].at[pl.ds(0, STRIDE)],
                            semw).wait()

                def scan_body(g, cnt_vec):
                    sg = seg_v[pl.ds(g * 16, 16)]
                    m = (sg >= lo) & (sg < lo + R)
                    pref = plsc.cumsum(m.astype(jnp.int32))
                    offs = jnp.where(m, cnt_vec + pref - 1, trash)
                    j = 0
                    if have0:
                        plsc.store_scatter(lbufs[j], [offs],
                                           pos0_v[pl.ds(g * 16, 16)])
                        j += 1
                    if want_iota:
                        plsc.store_scatter(lbufs[j], [offs],
                                           ebase + g * 16 + lane)
                        j += 1
                    plsc.store_scatter(lbufs[j], [offs], sg - lo)
                    return cnt_vec + plsc.all_reduce_population_count(m)

                cnt_vec = lax.fori_loop(0, NG, scan_body,
                                        jnp.zeros((16,), jnp.int32))
                for g in range(8):
                    pad_off = cnt_vec + g * 16 + lane
                    for j in range(nlists):
                        plsc.store_scatter(lbufs[j], [pad_off],
                                           jnp.zeros((16,), jnp.int32))
                    plsc.store_scatter(lbufs[nlists], [pad_off],
                                       jnp.full((16,), R, jnp.int32))
                cnt = jnp.max(cnt_vec, axis=0)
                nch = (cnt + 127) // 128
                coffs = jnp.where(lane == 0, kk, 48 + lane)
                plsc.store_scatter(counts_v, [coffs],
                                   jnp.full((16,), 1, jnp.int32) * nch)
                lbase = (rid * 16 + tid) * STRIDE
                for j in range(nlists + 1):
                    pltpu.async_copy(lbufs[j],
                                     out_lists[j].at[pl.ds(lbase, STRIDE)],
                                     semw)

        for b in lbufs:
            pltpu.make_async_copy(b, out_lists[0].at[pl.ds(0, STRIDE)],
                                  semw).wait()
        wid = tid * _NC + cid
        pltpu.sync_copy(counts_v.at[pl.ds(0, 32)],
                        counts_hbm.at[pl.ds(wid * 32, 32)])

    args = (seg,) + ((pos0,) if have0 else ())
    return k(*args)


def _sc_segsum_consume(table, lists_pos, lists_loc, counts, num_segments):
    D = table.shape[1]
    R, NR, S_pad, KMAX = _seg_geom(num_segments)
    K16 = lists_pos.shape[0] // (NR * 16)
    STRIDE = K16
    R16 = R // 16
    NZCH = R16 // 128
    NCHMAX = STRIDE // 128

    zeros_blk = jnp.zeros((128, _D), jnp.float32)
    mesh = plsc.VectorSubcoreMesh(core_axis_name="c", subcore_axis_name="s")

    @functools.partial(
        pl.kernel, mesh=mesh,
        compiler_params=pltpu.CompilerParams(needs_layout_passes=False),
        out_type=jax.ShapeDtypeStruct((S_pad, D), jnp.float32),
        scratch_types=[
            pltpu.VMEM((STRIDE,), jnp.int32),
            pltpu.VMEM((STRIDE,), jnp.int32),
            pltpu.VMEM((2, 128), jnp.int32),
            pltpu.VMEM((2, 128, D), jnp.float32),
            pltpu.VMEM((64,), jnp.int32),
            pltpu.VMEM_SHARED((_SEG_R + 8, _D), jnp.float32),
            pltpu.SemaphoreType.DMA,
            pltpu.SemaphoreType.DMA,
            pltpu.SemaphoreType.DMA,
        ],
    )
    def k(table_hbm, lpos_hbm, lloc_hbm, counts_hbm, zeros_hbm, out_hbm,
          pos_l, loc_l, loc2d, rows_v, counts_v, acc, sem0, sem1, semz):
        cid = lax.axis_index("c")
        tid = lax.axis_index("s")
        wid = tid * _NC + cid
        pltpu.sync_copy(counts_hbm.at[pl.ds(wid * 32, 32)],
                        counts_v.at[pl.ds(0, 32)])
        lane = lax.iota(jnp.int32, 16)
        gsem = (sem0, sem1)

        for kk in range(KMAX):
            rid = kk * 2 + cid

            @pl.when(rid < NR)
            def _range():
                lo = rid * R
                zh = [pltpu.async_copy(
                    zeros_hbm, acc.at[pl.ds(tid * R16 + zc * 128, 128)],
                    semz) for zc in range(NZCH)]
                lbase = (rid * 16 + tid) * STRIDE
                pltpu.sync_copy(lpos_hbm.at[pl.ds(lbase, STRIDE)], pos_l)
                pltpu.sync_copy(lloc_hbm.at[pl.ds(lbase, STRIDE)], loc_l)
                cv = counts_v[pl.ds((kk // 16) * 16, 16)]
                nch = jnp.max(jnp.where(lane == (kk % 16), cv, 0), axis=0)
                for h in zh:
                    h.wait()
                plsc.subcore_barrier()

                def fire_g(j, b):
                    pltpu.async_copy(
                        table_hbm.at[pos_l.at[pl.ds(j * 128, 128)]],
                        rows_v.at[b], gsem[b])

                def wait_g(j, b):
                    pltpu.make_async_copy(
                        table_hbm.at[pos_l.at[pl.ds(j * 128, 128)]],
                        rows_v.at[b], gsem[b]).wait()

                def do_scat(j, b):
                    for g in range(8):
                        loc2d[b, pl.ds(g * 16, 16)] = (
                            loc_l[pl.ds(j * 128 + g * 16, 16)])
                    pltpu.sync_copy(rows_v.at[b], acc.at[loc2d.at[b]],
                                    add=True)

                @pl.when(nch > 0)
                def _prime():
                    fire_g(0, 0)

                def pair_body(i, _):
                    j0 = 2 * i
                    j1 = j0 + 1

                    @pl.when(j1 < nch)
                    def _():
                        fire_g(j1, 1)

                    wait_g(j0, 0)
                    do_scat(j0, 0)

                    @pl.when(j1 < nch)
                    def _():
                        @pl.when(j1 + 1 < nch)
                        def _():
                            fire_g(j1 + 1, 0)

                        wait_g(j1, 1)
                        do_scat(j1, 1)

                    return 0

                lax.fori_loop(0, (nch + 1) // 2, pair_body, 0)
                plsc.subcore_barrier()
                for zc in range(NZCH):
                    pltpu.sync_copy(
                        acc.at[pl.ds(tid * R16 + zc * 128, 128)],
                        out_hbm.at[pl.ds(lo + tid * R16 + zc * 128, 128)])
                plsc.subcore_barrier()

    out = k(table, lists_pos, lists_loc, counts, zeros_blk)
    return out[:num_segments]


def _iota(n):
    return jnp.arange(n, dtype=jnp.int32)


def kernel(node, connect, bond, bond_neighbour, W_node_w, W_node_b,
           W_node_final_w, W_node_final_b, W_bond_w, W_bond_b,
           W_bond_final_w, W_bond_final_b, W_z_w, W_z_b, W_r_w, W_r_b,
           U_w, W_w, W_b, W_n_w, W_n_b, U_n_w):
    i_idx = connect[0]
    j_idx = connect[1]
    ij_idx = bond_neighbour[0]
    ki_idx = bond_neighbour[1]
    N = node.shape[0]
    E = bond.shape[0]
    FN = node.shape[1]
    FB = bond.shape[1]

    nodei = _gather_rows(node, i_idx)

    mess_bond = _mm_fused([nodei, bond], [W_bond_w[:FN], W_bond_w[FN:]],
                          W_bond_b, "hswish")
    mess_node = _mm_fused([node], [W_node_w], W_node_b, "hswish",
                          block_rows=2000)
    pre_z = _mm_fused([nodei, bond], [W_z_w[:FN], W_z_w[FN:FN + FB]],
                      W_z_b, "none")
    pre_m = _mm_fused([nodei, bond], [W_w[:FN], W_w[FN:]], W_b, "none")
    pre_n = _mm_fused([node], [W_n_w], W_n_b, "none", block_rows=2000)

    pre_r_e = _mm_fused([nodei, bond], [W_r_w[:FN], W_r_w[FN:FN + FB]],
                        W_r_b, "none")
    pre_r = _gather_rows(pre_r_e, ij_idx)

    wz2 = W_z_w[FN + FB:]
    wr2 = W_r_w[FN + FB:]
    un1 = U_n_w[:_D]
    un2 = U_n_w[_D:]

    iota_enb = _iota(ij_idx.shape[0])
    iota_e = _iota(E)
    for _ in range(_LAYER):
        s_ij = _sc_segsum_gather(mess_bond, ki_idx, ij_idx, E)
        mess_ki = _gather_rows(mess_bond, ki_idx)
        rmk = _rki_fused(pre_r, mess_ki, wr2)
        r_ij = _sc_segsum_gather(rmk, iota_enb, ij_idx, E)
        mess_bond = _bond_upd(pre_z, pre_m, s_ij, r_ij, wz2, U_w)
        aggr_node = _sc_segsum_gather(mess_bond, iota_e, j_idx, N)
        mess_node = _node_upd(pre_n, mess_node, aggr_node, un1, un2)

    out_bond = _mm_fused([nodei, bond, mess_bond],
                         [W_bond_final_w[:FN], W_bond_final_w[FN:FN + FB],
                          W_bond_final_w[FN + FB:]],
                         W_bond_final_b, "hswish")
    out_node = _mm_fused([node, mess_node],
                         [W_node_final_w[:FN], W_node_final_w[FN:]],
                         W_node_final_b, "hswish", block_rows=2000)
    return (out_node, out_bond)


# speedup vs baseline: 1.4396x; 1.0006x over previous
import functools

import jax
import jax.numpy as jnp
from jax import lax
from jax.experimental import pallas as pl
from jax.experimental.pallas import tpu as pltpu
from jax.experimental.pallas import tpu_sc as plsc

_LAYER = 3
_D = 128
_NC, _NS = 2, 16
_NW = _NC * _NS


def _hswish(x):
    return x * jnp.clip(x + 3.0, 0.0, 6.0) / 6.0


def _mm_body(act, nx, *refs):
    in_refs = refs[:nx]
    w_refs = refs[nx:2 * nx]
    b_ref = refs[2 * nx]
    o_ref = refs[2 * nx + 1]
    acc = b_ref[...].astype(jnp.float32)
    for x_ref, w_ref in zip(in_refs, w_refs):
        acc = acc + jnp.dot(x_ref[...], w_ref[...],
                            preferred_element_type=jnp.float32)
    if act == "hswish":
        acc = _hswish(acc)
    elif act == "sigmoid":
        acc = jax.nn.sigmoid(acc)
    o_ref[...] = acc


def _mm_fused(xs, ws, bias, act, block_rows=2000):
    rows = xs[0].shape[0]
    grid = (rows // block_rows,)
    nx = len(xs)
    in_specs = (
        [pl.BlockSpec((block_rows, x.shape[1]), lambda i: (i, 0)) for x in xs]
        + [pl.BlockSpec(w.shape, lambda i: (0, 0)) for w in ws]
        + [pl.BlockSpec((1, _D), lambda i: (0, 0))]
    )
    return pl.pallas_call(
        functools.partial(_mm_body, act, nx),
        grid=grid,
        in_specs=in_specs,
        out_specs=pl.BlockSpec((block_rows, _D), lambda i: (i, 0)),
        out_shape=jax.ShapeDtypeStruct((rows, _D), jnp.float32),
    )(*xs, *ws, bias.reshape(1, _D))


def _rki_body(pre_ref, mk_ref, w_ref, o_ref):
    mk = mk_ref[...]
    r = jax.nn.sigmoid(pre_ref[...] + jnp.dot(mk, w_ref[...],
                                              preferred_element_type=jnp.float32))
    o_ref[...] = r * mk


def _rki_fused(pre, mk, w, block_rows=2000):
    rows = pre.shape[0]
    return pl.pallas_call(
        _rki_body,
        grid=(rows // block_rows,),
        in_specs=[
            pl.BlockSpec((block_rows, _D), lambda i: (i, 0)),
            pl.BlockSpec((block_rows, _D), lambda i: (i, 0)),
            pl.BlockSpec((_D, _D), lambda i: (0, 0)),
        ],
        out_specs=pl.BlockSpec((block_rows, _D), lambda i: (i, 0)),
        out_shape=jax.ShapeDtypeStruct((rows, _D), jnp.float32),
    )(pre, mk, w)


def _bond_upd_body(pre_z_ref, pre_m_ref, s_ref, r_ref, wz_ref, uw_ref, o_ref):
    s = s_ref[...]
    z = jax.nn.sigmoid(pre_z_ref[...] + jnp.dot(s, wz_ref[...],
                                                preferred_element_type=jnp.float32))
    m = jnp.tanh(pre_m_ref[...] + jnp.dot(r_ref[...], uw_ref[...],
                                          preferred_element_type=jnp.float32))
    o_ref[...] = (1.0 - z) * s + z * m


def _bond_upd(pre_z, pre_m, s, r, wz, uw, block_rows=2000):
    rows = pre_z.shape[0]
    bs = lambda: pl.BlockSpec((block_rows, _D), lambda i: (i, 0))
    return pl.pallas_call(
        _bond_upd_body,
        grid=(rows // block_rows,),
        in_specs=[bs(), bs(), bs(), bs(),
                  pl.BlockSpec((_D, _D), lambda i: (0, 0)),
                  pl.BlockSpec((_D, _D), lambda i: (0, 0))],
        out_specs=bs(),
        out_shape=jax.ShapeDtypeStruct((rows, _D), jnp.float32),
    )(pre_z, pre_m, s, r, wz, uw)


def _node_upd_body(pre_ref, mn_ref, ag_ref, u1_ref, u2_ref, o_ref):
    acc = pre_ref[...]
    acc = acc + jnp.dot(mn_ref[...], u1_ref[...], preferred_element_type=jnp.float32)
    acc = acc + jnp.dot(ag_ref[...], u2_ref[...], preferred_element_type=jnp.float32)
    o_ref[...] = _hswish(acc)


def _node_upd(pre_n, mn, aggr, u1, u2, block_rows=2000):
    rows = pre_n.shape[0]
    bs = lambda: pl.BlockSpec((block_rows, _D), lambda i: (i, 0))
    return pl.pallas_call(
        _node_upd_body,
        grid=(rows // block_rows,),
        in_specs=[bs(), bs(), bs(),
                  pl.BlockSpec((_D, _D), lambda i: (0, 0)),
                  pl.BlockSpec((_D, _D), lambda i: (0, 0))],
        out_specs=bs(),
        out_shape=jax.ShapeDtypeStruct((rows, _D), jnp.float32),
    )(pre_n, mn, aggr, u1, u2)


def _sc_gather(table, idx):
    K = idx.shape[0]
    D = table.shape[1]
    PW = K // _NW
    assert K % _NW == 0 and PW % 8 == 0, (K, PW)
    CH = min(128, PW)
    NFULL = PW // CH
    TAIL = PW - NFULL * CH
    assert TAIL % 8 == 0

    mesh = plsc.VectorSubcoreMesh(core_axis_name="c", subcore_axis_name="s")

    @functools.partial(
        pl.kernel, mesh=mesh,
        out_type=jax.ShapeDtypeStruct((K, D), jnp.float32),
        scratch_types=[
            pltpu.VMEM((PW,), jnp.int32),
            pltpu.VMEM((2, CH, D), jnp.float32),
            pltpu.SemaphoreType.DMA,
            pltpu.SemaphoreType.DMA,
        ],
    )
    def k(table_hbm, idx_hbm, out_hbm, idx_v, rows_v, sem0, sem1):
        wid = lax.axis_index("s") * _NC + lax.axis_index("c")
        base = wid * PW
        pltpu.sync_copy(idx_hbm.at[pl.ds(base, PW)], idx_v)
        sems = (sem0, sem1)

        def fire(c, b):
            pltpu.async_copy(table_hbm.at[idx_v.at[pl.ds(c * CH, CH)]],
                             rows_v.at[b], sems[b])

        def wait_write(c, b):
            pltpu.make_async_copy(
                table_hbm.at[idx_v.at[pl.ds(c * CH, CH)]],
                rows_v.at[b], sems[b]).wait()
            pltpu.sync_copy(rows_v.at[b],
                            out_hbm.at[pl.ds(base + c * CH, CH)])

        fire(0, 0)
        for c in range(1, NFULL):
            fire(c, c & 1)
            wait_write(c - 1, (c - 1) & 1)
        wait_write(NFULL - 1, (NFULL - 1) & 1)
        if TAIL:
            pltpu.async_copy(
                table_hbm.at[idx_v.at[pl.ds(NFULL * CH, TAIL)]],
                rows_v.at[1, pl.ds(0, TAIL)], sem1).wait()
            pltpu.sync_copy(rows_v.at[1, pl.ds(0, TAIL)],
                            out_hbm.at[pl.ds(base + NFULL * CH, TAIL)])

    return k(table, idx)


def _gather_rows(table, idx):
    return _sc_gather(table, idx)


_SEG_R = 8192
_CH = 64


def _sc_segsum_gather(table, pos, seg, num_segments):
    K = seg.shape[0]
    D = table.shape[1]
    assert D == _D
    R = min(_SEG_R, ((num_segments + 4095) // 4096) * 2048)
    NR = (num_segments + R - 1) // R
    S_pad = NR * R
    R16 = R // 16
    NZCH = R16 // 128
    assert R16 % 128 == 0
    PS = K // 16
    NG = PS // 16
    assert K % 256 == 0
    LCAP = PS + 144
    KMAX = (NR + 1) // 2

    zeros_blk = jnp.zeros((128, _D), jnp.float32)
    mesh = plsc.VectorSubcoreMesh(core_axis_name="c", subcore_axis_name="s")

    @functools.partial(
        pl.kernel, mesh=mesh,
        compiler_params=pltpu.CompilerParams(needs_layout_passes=False),
        out_type=jax.ShapeDtypeStruct((S_pad, D), jnp.float32),
        scratch_types=[
            pltpu.VMEM((PS,), jnp.int32),
            pltpu.VMEM((PS,), jnp.int32),
            pltpu.VMEM((LCAP,), jnp.int32),
            pltpu.VMEM((LCAP,), jnp.int32),
            pltpu.VMEM((2, _CH), jnp.int32),
            pltpu.VMEM((2, _CH, D), jnp.float32),
            pltpu.VMEM_SHARED((_SEG_R + 8, _D), jnp.float32),
            pltpu.SemaphoreType.DMA,
            pltpu.SemaphoreType.DMA,
            pltpu.SemaphoreType.DMA,
        ],
    )
    def k(table_hbm, pos_hbm, seg_hbm, zeros_hbm, out_hbm,
          seg_v, pos_v, pos_l, loc_l, loc2d, rows_v, acc,
          sem0, sem1, semz):
        cid = lax.axis_index("c")
        tid = lax.axis_index("s")
        ebase = tid * PS
        pltpu.sync_copy(seg_hbm.at[pl.ds(ebase, PS)], seg_v)
        pltpu.sync_copy(pos_hbm.at[pl.ds(ebase, PS)], pos_v)
        gsem = (sem0, sem1)

        for kk in range(KMAX):
            rid = kk * 2 + cid

            @pl.when(rid < NR)
            def _range():
                lo = rid * R
                zh = [pltpu.async_copy(
                    zeros_hbm, acc.at[pl.ds(tid * R16 + zc * 128, 128)],
                    semz) for zc in range(NZCH)]

                lane = lax.iota(jnp.int32, 16)
                trash = jnp.full((16,), PS + 128, jnp.int32) + lane

                def scan_body(g, cnt_vec):
                    sg = seg_v[pl.ds(g * 16, 16)]
                    m = (sg >= lo) & (sg < lo + R)
                    pref = plsc.cumsum(m.astype(jnp.int32))
                    offs = jnp.where(m, cnt_vec + pref - 1, trash)
                    plsc.store_scatter(pos_l, [offs],
                                       pos_v[pl.ds(g * 16, 16)])
                    plsc.store_scatter(loc_l, [offs], sg - lo)
                    return cnt_vec + plsc.all_reduce_population_count(m)

                cnt_vec = lax.fori_loop(0, NG, scan_body,
                                        jnp.zeros((16,), jnp.int32))
                for g in range(_CH // 16):
                    pad_off = cnt_vec + g * 16 + lane
                    plsc.store_scatter(pos_l, [pad_off],
                                       jnp.zeros((16,), jnp.int32))
                    plsc.store_scatter(loc_l, [pad_off],
                                       jnp.full((16,), R, jnp.int32))
                cnt = jnp.max(cnt_vec, axis=0)
                nch = (cnt + _CH - 1) // _CH
                for h in zh:
                    h.wait()
                plsc.subcore_barrier()

                def fire_g(j, b):
                    pltpu.async_copy(
                        table_hbm.at[pos_l.at[pl.ds(j * _CH, _CH)]],
                        rows_v.at[b], gsem[b])

                def wait_g(j, b):
                    pltpu.make_async_copy(
                        table_hbm.at[pos_l.at[pl.ds(j * _CH, _CH)]],
                        rows_v.at[b], gsem[b]).wait()

                def do_scat(j, b):
                    for g in range(_CH // 16):
                        loc2d[b, pl.ds(g * 16, 16)] = (
                            loc_l[pl.ds(j * _CH + g * 16, 16)])
                    pltpu.sync_copy(rows_v.at[b], acc.at[loc2d.at[b]],
                                    add=True)

                @pl.when(nch > 0)
                def _prime():
                    fire_g(0, 0)

                def pair_body(i, _):
                    j0 = 2 * i
                    j1 = j0 + 1

                    @pl.when(j1 < nch)
                    def _():
                        fire_g(j1, 1)

                    wait_g(j0, 0)
                    do_scat(j0, 0)

                    @pl.when(j1 < nch)
                    def _():
                        @pl.when(j1 + 1 < nch)
                        def _():
                            fire_g(j1 + 1, 0)

                        wait_g(j1, 1)
                        do_scat(j1, 1)

                    return 0

                lax.fori_loop(0, (nch + 1) // 2, pair_body, 0)
                plsc.subcore_barrier()

                for zc in range(NZCH):
                    pltpu.sync_copy(
                        acc.at[pl.ds(tid * R16 + zc * 128, 128)],
                        out_hbm.at[pl.ds(lo + tid * R16 + zc * 128, 128)])

    out = k(table, pos, seg, zeros_blk)
    return out[:num_segments]


def _iota(n):
    return jnp.arange(n, dtype=jnp.int32)


def kernel(node, connect, bond, bond_neighbour, W_node_w, W_node_b,
           W_node_final_w, W_node_final_b, W_bond_w, W_bond_b,
           W_bond_final_w, W_bond_final_b, W_z_w, W_z_b, W_r_w, W_r_b,
           U_w, W_w, W_b, W_n_w, W_n_b, U_n_w):
    i_idx = connect[0]
    j_idx = connect[1]
    ij_idx = bond_neighbour[0]
    ki_idx = bond_neighbour[1]
    N = node.shape[0]
    E = bond.shape[0]
    FN = node.shape[1]
    FB = bond.shape[1]

    nodei = _gather_rows(node, i_idx)

    mess_bond = _mm_fused([nodei, bond], [W_bond_w[:FN], W_bond_w[FN:]],
                          W_bond_b, "hswish")
    mess_node = _mm_fused([node], [W_node_w], W_node_b, "hswish",
                          block_rows=2000)
    pre_z = _mm_fused([nodei, bond], [W_z_w[:FN], W_z_w[FN:FN + FB]],
                      W_z_b, "none")
    pre_m = _mm_fused([nodei, bond], [W_w[:FN], W_w[FN:]], W_b, "none")
    pre_n = _mm_fused([node], [W_n_w], W_n_b, "none", block_rows=2000)

    pre_r_e = _mm_fused([nodei, bond], [W_r_w[:FN], W_r_w[FN:FN + FB]],
                        W_r_b, "none")
    pre_r = _gather_rows(pre_r_e, ij_idx)

    wz2 = W_z_w[FN + FB:]
    wr2 = W_r_w[FN + FB:]
    un1 = U_n_w[:_D]
    un2 = U_n_w[_D:]

    iota_enb = _iota(ij_idx.shape[0])
    iota_e = _iota(E)
    for _ in range(_LAYER):
        s_ij = _sc_segsum_gather(mess_bond, ki_idx, ij_idx, E)
        mess_ki = _gather_rows(mess_bond, ki_idx)
        rmk = _rki_fused(pre_r, mess_ki, wr2)
        r_ij = _sc_segsum_gather(rmk, iota_enb, ij_idx, E)
        mess_bond = _bond_upd(pre_z, pre_m, s_ij, r_ij, wz2, U_w)
        aggr_node = _sc_segsum_gather(mess_bond, iota_e, j_idx, N)
        mess_node = _node_upd(pre_n, mess_node, aggr_node, un1, un2)

    out_bond = _mm_fused([nodei, bond, mess_bond],
                         [W_bond_final_w[:FN], W_bond_final_w[FN:FN + FB],
                          W_bond_final_w[FN + FB:]],
                         W_bond_final_b, "hswish")
    out_node = _mm_fused([node, mess_node],
                         [W_node_final_w[:FN], W_node_final_w[FN:]],
                         W_node_final_b, "hswish", block_rows=2000)
    return (out_node, out_bond)
